# baseline (speedup 1.0000x reference)
import functools

import jax
import jax.numpy as jnp
from jax import lax
from jax.experimental import pallas as pl
from jax.experimental.pallas import tpu as pltpu
from jax.experimental.pallas import tpu_sc as plsc

PAD, BOS, EOS = 0, 1, 2
NC, NS, L = 2, 16, 16
ROWS_PER_CORE = 8
CH = 64
NB = 2


def _body(B, T, S, tok_hbm, sco_hbm, attn_hbm, wdp_hbm,
          otok_hbm, osco_hbm, oattn_hbm,
          tok_v, wdp_v, sco_v, src_v, otok_v, osco_v, nk_v, src2_v,
          gb0, gb1, zbuf, spm_src, spm_nk, sem, zsem,
          gs0, gs1, ss0, ss1):
    gbufs = [gb0, gb1]
    gsems = [gs0, gs1]
    ssems = [ss0, ss1]
    c = lax.axis_index("c")
    s = lax.axis_index("s")
    half = T // 2
    nch = half // CH

    @pl.loop(0, CH)
    def _(r):
        @pl.loop(0, S // L)
        def _(cc):
            zbuf[r, pl.ds(cc * L, L)] = jnp.zeros((L,), jnp.float32)

    @pl.when(s < ROWS_PER_CORE)
    def _():
        b = c * ROWS_PER_CORE + s
        pltpu.sync_copy(tok_hbm.at[b], tok_v)
        pltpu.sync_copy(wdp_hbm.at[b], wdp_v)

        row0 = b * T

        @pl.loop(0, T // L, unroll=4)
        def _(i):
            src_v[pl.ds(i * L, L)] = jnp.full((L,), row0, jnp.int32)

        @pl.loop(0, T // L, init_carry=jnp.zeros((L,), jnp.int32), unroll=4)
        def nk_vec(i, carry):
            base = i * L
            tok = tok_v[pl.ds(base, L)]
            wdp = wdp_v[pl.ds(base, L)] != 0
            pad_mask = tok == PAD
            boseos = (tok == BOS) | (tok == EOS)
            keep = boseos | jnp.logical_not(wdp | pad_mask)
            keep_i = jnp.where(keep, jnp.full((L,), 1, jnp.int32),
                               jnp.full((L,), 0, jnp.int32))
            cs = plsc.cumsum(keep_i)
            dest = carry + cs - keep_i
            pos = row0 + base + lax.iota(jnp.int32, L)
            plsc.store_scatter(src_v, [dest], pos, mask=keep)
            return carry + plsc.all_reduce_population_count(keep)

        nk_v[pl.ds(0, L)] = nk_vec
        pltpu.sync_copy(src_v, spm_src.at[s])
        pltpu.sync_copy(nk_v, spm_nk.at[s, pl.ds(0, L)])

    plsc.subcore_barrier()

    s8 = lax.rem(s, ROWS_PER_CORE)
    h = lax.div(s, ROWS_PER_CORE)
    b2 = c * ROWS_PER_CORE + s8
    pltpu.sync_copy(spm_src.at[s8], src2_v)
    pltpu.sync_copy(spm_nk.at[s8, pl.ds(0, L)], nk_v)
    nk = jnp.max(nk_v[...])
    out_base = b2 * T

    nchr = T // CH
    nf_row = lax.div(nk, CH)
    krem = lax.rem(nk, CH)
    zc = lax.div(nk + (CH - 1), CH)
    nfm = lax.div(nf_row - h + 1, 2)
    jz0 = lax.div(zc - h + 1, 2)

    @pl.loop(jz0, nchr // 2)
    def _(j):
        zc0 = (2 * j + h) * CH
        pltpu.async_copy(zbuf, oattn_hbm.at[pl.ds(out_base + zc0, CH)], zsem)

    for b in range(NB):
        @pl.when(b < nfm)
        def _(b=b):
            c0 = pl.multiple_of((2 * b + h) * CH, CH)
            pltpu.async_copy(
                attn_hbm.at[src2_v.at[pl.ds(c0, CH)]], gbufs[b], gsems[b])

    @pl.when(s < ROWS_PER_CORE)
    def _():
        b = c * ROWS_PER_CORE + s
        row0 = b * T
        pltpu.sync_copy(sco_hbm.at[b], sco_v)

        @pl.loop(0, T // L, unroll=4)
        def _(i):
            base = i * L
            lsrc = src_v[pl.ds(base, L)] - row0
            g_tok = plsc.load_gather(tok_v, [lsrc])
            g_sco = plsc.load_gather(sco_v, [lsrc])
            sel = (base + lax.iota(jnp.int32, L)) < nk
            otok_v[pl.ds(base, L)] = jnp.where(sel, g_tok, jnp.full((L,), PAD, jnp.int32))
            osco_v[pl.ds(base, L)] = jnp.where(sel, g_sco, jnp.zeros((L,), jnp.float32))

        pltpu.sync_copy(otok_v, otok_hbm.at[b])
        pltpu.sync_copy(osco_v, osco_hbm.at[b])

    ngroups = lax.div(nfm + (NB - 1), NB)

    @pl.loop(0, ngroups)
    def _(g):
        for b in range(NB):
            j = g * NB + b

            @pl.when(j < nfm)
            def _(b=b, j=j):
                c0 = pl.multiple_of((2 * j + h) * CH, CH)
                pltpu.make_async_copy(
                    attn_hbm.at[pl.ds(0, CH)], gbufs[b], gsems[b]).wait()
                pltpu.async_copy(
                    gbufs[b], oattn_hbm.at[pl.ds(out_base + c0, CH)], ssems[b])
                jn = j + NB

                @pl.when(jn < nfm)
                def _():
                    pltpu.make_async_copy(
                        gbufs[b], oattn_hbm.at[pl.ds(out_base, CH)],
                        ssems[b]).wait()
                    cn = pl.multiple_of((2 * jn + h) * CH, CH)
                    pltpu.async_copy(
                        attn_hbm.at[src2_v.at[pl.ds(cn, CH)]], gbufs[b],
                        gsems[b])

    for b in range(NB):
        @pl.when(b < nfm)
        def _(b=b):
            pltpu.make_async_copy(
                gbufs[b], oattn_hbm.at[pl.ds(out_base, CH)], ssems[b]).wait()

    @pl.when((krem != 0) & (lax.rem(nf_row, 2) == h))
    def _():
        pc0 = pl.multiple_of(nf_row * CH, CH)
        cp = pltpu.async_copy(attn_hbm.at[src2_v.at[pl.ds(pc0, CH)]], gbufs[0], sem)
        cp.wait()

        @pl.loop(krem, CH)
        def _(r):
            @pl.loop(0, S // L)
            def _(cc):
                gbufs[0][r, pl.ds(cc * L, L)] = jnp.zeros((L,), jnp.float32)

        pltpu.sync_copy(gbufs[0], oattn_hbm.at[pl.ds(out_base + pc0, CH)])

    @pl.loop(jz0, nchr // 2)
    def _(j):
        pltpu.make_async_copy(
            zbuf, oattn_hbm.at[pl.ds(out_base, CH)], zsem).wait()


def kernel(in_tokens, in_scores, in_attn, word_del_pred):
    B, T = in_tokens.shape
    S = in_attn.shape[-1]
    tok = in_tokens.astype(jnp.int32)
    wdp = word_del_pred.astype(jnp.int32)
    attn2 = in_attn.reshape(B * T, S)

    mesh = plsc.VectorSubcoreMesh(
        core_axis_name="c", subcore_axis_name="s", num_cores=NC, num_subcores=NS
    )
    half = T // 2
    kfn = pl.kernel(
        functools.partial(_body, B, T, S),
        out_type=[
            jax.ShapeDtypeStruct((B, T), jnp.int32),
            jax.ShapeDtypeStruct((B, T), jnp.float32),
            jax.ShapeDtypeStruct((B * T, S), jnp.float32),
        ],
        mesh=mesh,
        compiler_params=pltpu.CompilerParams(needs_layout_passes=False),
        scratch_types=[
            pltpu.VMEM((T,), jnp.int32),
            pltpu.VMEM((T,), jnp.int32),
            pltpu.VMEM((T,), jnp.float32),
            pltpu.VMEM((T,), jnp.int32),
            pltpu.VMEM((T,), jnp.int32),
            pltpu.VMEM((T,), jnp.float32),
            pltpu.VMEM((L,), jnp.int32),
            pltpu.VMEM((T,), jnp.int32),
            pltpu.VMEM((CH, S), jnp.float32),
            pltpu.VMEM((CH, S), jnp.float32),
            pltpu.VMEM((CH, S), jnp.float32),
            pltpu.VMEM_SHARED((ROWS_PER_CORE, T), jnp.int32),
            pltpu.VMEM_SHARED((ROWS_PER_CORE, 128), jnp.int32),
            pltpu.SemaphoreType.DMA,
            pltpu.SemaphoreType.DMA,
            pltpu.SemaphoreType.DMA,
            pltpu.SemaphoreType.DMA,
            pltpu.SemaphoreType.DMA,
            pltpu.SemaphoreType.DMA,
        ],
    )
    otok, osco, oattn = kfn(tok, in_scores, attn2, wdp)
    return (otok, osco, oattn.reshape(B, T, S))


# speedup vs baseline: 1.1552x; 1.1552x over previous
import functools

import jax
import jax.numpy as jnp
from jax import lax
from jax.experimental import pallas as pl
from jax.experimental.pallas import tpu as pltpu
from jax.experimental.pallas import tpu_sc as plsc

PAD, BOS, EOS = 0, 1, 2
NC, NS, L = 2, 16, 16
ROWS_PER_CORE = 8
CH = 32
NB = 5


def _body(B, T, S, tok_hbm, sco_hbm, attn_hbm, wdp_hbm,
          otok_hbm, osco_hbm, oattn_hbm,
          tok_v, wdp_v, sco_v, src_v, otok_v, osco_v, nk_v, src2_v,
          gb0, gb1, gb2, gb3, gb4, zbuf, spm_src, spm_nk, sem, zsem,
          gs0, gs1, gs2, gs3, gs4, ss0, ss1, ss2, ss3, ss4):
    gbufs = [gb0, gb1, gb2, gb3, gb4]
    gsems = [gs0, gs1, gs2, gs3, gs4]
    ssems = [ss0, ss1, ss2, ss3, ss4]
    c = lax.axis_index("c")
    s = lax.axis_index("s")
    half = T // 2
    nch = half // CH

    @pl.loop(0, CH)
    def _(r):
        @pl.loop(0, S // L)
        def _(cc):
            zbuf[r, pl.ds(cc * L, L)] = jnp.zeros((L,), jnp.float32)

    @pl.when(s < ROWS_PER_CORE)
    def _():
        b = c * ROWS_PER_CORE + s
        cp_t = pltpu.async_copy(tok_hbm.at[b], tok_v, sem)
        cp_w = pltpu.async_copy(wdp_hbm.at[b], wdp_v, zsem)
        cp_t.wait()
        cp_w.wait()

        row0 = b * T

        @pl.loop(0, T // L, unroll=4)
        def _(i):
            src_v[pl.ds(i * L, L)] = jnp.full((L,), row0, jnp.int32)

        @pl.loop(0, T // L, init_carry=jnp.zeros((L,), jnp.int32), unroll=4)
        def nk_vec(i, carry):
            base = i * L
            tok = tok_v[pl.ds(base, L)]
            wdp = wdp_v[pl.ds(base, L)] != 0
            pad_mask = tok == PAD
            boseos = (tok == BOS) | (tok == EOS)
            keep = boseos | jnp.logical_not(wdp | pad_mask)
            keep_i = jnp.where(keep, jnp.full((L,), 1, jnp.int32),
                               jnp.full((L,), 0, jnp.int32))
            cs = plsc.cumsum(keep_i)
            dest = carry + cs - keep_i
            pos = row0 + base + lax.iota(jnp.int32, L)
            plsc.store_scatter(src_v, [dest], pos, mask=keep)
            return carry + plsc.all_reduce_population_count(keep)

        nk_v[pl.ds(0, L)] = nk_vec
        pltpu.sync_copy(src_v, spm_src.at[s])
        pltpu.sync_copy(nk_v, spm_nk.at[s, pl.ds(0, L)])

    plsc.subcore_barrier()

    s8 = lax.rem(s, ROWS_PER_CORE)
    h = lax.div(s, ROWS_PER_CORE)
    b2 = c * ROWS_PER_CORE + s8
    pltpu.sync_copy(spm_src.at[s8], src2_v)
    pltpu.sync_copy(spm_nk.at[s8, pl.ds(0, L)], nk_v)
    nk = jnp.max(nk_v[...])
    out_base = b2 * T

    nchr = T // CH
    nf_row = lax.div(nk, CH)
    krem = lax.rem(nk, CH)
    zc = lax.div(nk + (CH - 1), CH)
    nfm = lax.div(nf_row - h + 1, 2)
    jz0 = lax.div(zc - h + 1, 2)

    @pl.loop(jz0, nchr // 2)
    def _(j):
        zc0 = (2 * j + h) * CH
        pltpu.async_copy(zbuf, oattn_hbm.at[pl.ds(out_base + zc0, CH)], zsem)

    for b in range(NB):
        @pl.when(b < nfm)
        def _(b=b):
            c0 = pl.multiple_of((2 * b + h) * CH, CH)
            pltpu.async_copy(
                attn_hbm.at[src2_v.at[pl.ds(c0, CH)]], gbufs[b], gsems[b])

    @pl.when(s < ROWS_PER_CORE)
    def _():
        b = c * ROWS_PER_CORE + s
        row0 = b * T
        pltpu.sync_copy(sco_hbm.at[b], sco_v)

        @pl.loop(0, T // L, unroll=4)
        def _(i):
            base = i * L
            lsrc = src_v[pl.ds(base, L)] - row0
            g_tok = plsc.load_gather(tok_v, [lsrc])
            g_sco = plsc.load_gather(sco_v, [lsrc])
            sel = (base + lax.iota(jnp.int32, L)) < nk
            otok_v[pl.ds(base, L)] = jnp.where(sel, g_tok, jnp.full((L,), PAD, jnp.int32))
            osco_v[pl.ds(base, L)] = jnp.where(sel, g_sco, jnp.zeros((L,), jnp.float32))

        pltpu.sync_copy(otok_v, otok_hbm.at[b])
        pltpu.sync_copy(osco_v, osco_hbm.at[b])

    ngroups = lax.div(nfm + (NB - 1), NB)

    @pl.loop(0, ngroups)
    def _(g):
        for b in range(NB):
            j = g * NB + b

            @pl.when(j < nfm)
            def _(b=b, j=j):
                c0 = pl.multiple_of((2 * j + h) * CH, CH)
                pltpu.make_async_copy(
                    attn_hbm.at[pl.ds(0, CH)], gbufs[b], gsems[b]).wait()
                pltpu.async_copy(
                    gbufs[b], oattn_hbm.at[pl.ds(out_base + c0, CH)], ssems[b])
                jn = j + NB

                @pl.when(jn < nfm)
                def _():
                    pltpu.make_async_copy(
                        gbufs[b], oattn_hbm.at[pl.ds(out_base, CH)],
                        ssems[b]).wait()
                    cn = pl.multiple_of((2 * jn + h) * CH, CH)
                    pltpu.async_copy(
                        attn_hbm.at[src2_v.at[pl.ds(cn, CH)]], gbufs[b],
                        gsems[b])

    for b in range(NB):
        @pl.when(b < nfm)
        def _(b=b):
            pltpu.make_async_copy(
                gbufs[b], oattn_hbm.at[pl.ds(out_base, CH)], ssems[b]).wait()

    @pl.when((krem != 0) & (lax.rem(nf_row, 2) == h))
    def _():
        pc0 = pl.multiple_of(nf_row * CH, CH)
        cp = pltpu.async_copy(attn_hbm.at[src2_v.at[pl.ds(pc0, CH)]], gbufs[0], sem)
        cp.wait()

        @pl.loop(krem, CH)
        def _(r):
            @pl.loop(0, S // L)
            def _(cc):
                gbufs[0][r, pl.ds(cc * L, L)] = jnp.zeros((L,), jnp.float32)

        pltpu.sync_copy(gbufs[0], oattn_hbm.at[pl.ds(out_base + pc0, CH)])

    @pl.loop(jz0, nchr // 2)
    def _(j):
        pltpu.make_async_copy(
            zbuf, oattn_hbm.at[pl.ds(out_base, CH)], zsem).wait()


def kernel(in_tokens, in_scores, in_attn, word_del_pred):
    B, T = in_tokens.shape
    S = in_attn.shape[-1]
    tok = in_tokens.astype(jnp.int32)
    wdp = word_del_pred.astype(jnp.int32)
    attn2 = in_attn.reshape(B * T, S)

    mesh = plsc.VectorSubcoreMesh(
        core_axis_name="c", subcore_axis_name="s", num_cores=NC, num_subcores=NS
    )
    half = T // 2
    kfn = pl.kernel(
        functools.partial(_body, B, T, S),
        out_type=[
            jax.ShapeDtypeStruct((B, T), jnp.int32),
            jax.ShapeDtypeStruct((B, T), jnp.float32),
            jax.ShapeDtypeStruct((B * T, S), jnp.float32),
        ],
        mesh=mesh,
        compiler_params=pltpu.CompilerParams(needs_layout_passes=False),
        scratch_types=[
            pltpu.VMEM((T,), jnp.int32),
            pltpu.VMEM((T,), jnp.int32),
            pltpu.VMEM((T,), jnp.float32),
            pltpu.VMEM((T,), jnp.int32),
            pltpu.VMEM((T,), jnp.int32),
            pltpu.VMEM((T,), jnp.float32),
            pltpu.VMEM((L,), jnp.int32),
            pltpu.VMEM((T,), jnp.int32),
            pltpu.VMEM((CH, S), jnp.float32),
            pltpu.VMEM((CH, S), jnp.float32),
            pltpu.VMEM((CH, S), jnp.float32),
            pltpu.VMEM((CH, S), jnp.float32),
            pltpu.VMEM((CH, S), jnp.float32),
            pltpu.VMEM((CH, S), jnp.float32),
            pltpu.VMEM_SHARED((ROWS_PER_CORE, T), jnp.int32),
            pltpu.VMEM_SHARED((ROWS_PER_CORE, 128), jnp.int32),
            pltpu.SemaphoreType.DMA,
            pltpu.SemaphoreType.DMA,
            pltpu.SemaphoreType.DMA,
            pltpu.SemaphoreType.DMA,
            pltpu.SemaphoreType.DMA,
            pltpu.SemaphoreType.DMA,
            pltpu.SemaphoreType.DMA,
            pltpu.SemaphoreType.DMA,
            pltpu.SemaphoreType.DMA,
            pltpu.SemaphoreType.DMA,
            pltpu.SemaphoreType.DMA,
            pltpu.SemaphoreType.DMA,
        ],
    )
    otok, osco, oattn = kfn(tok, in_scores, attn2, wdp)
    return (otok, osco, oattn.reshape(B, T, S))
